# baseline (speedup 1.0000x reference)
import functools

import jax
import jax.numpy as jnp
from jax import lax
from jax.experimental import pallas as pl
from jax.experimental.pallas import tpu as pltpu
from jax.experimental.pallas import tpu_sc as plsc

B = 4096
L = 200
D = 64
N_CORES = 2
N_SUBCORES = 16
N_WORKERS = N_CORES * N_SUBCORES
S_PER_W = B // N_WORKERS
IDX_PER_W = S_PER_W * L
G0 = 128
G1 = L - G0


def _pool_body(i2_hbm, cb_hbm, t2_hbm, out_hbm, idx_v, cb_v, rows_v, pooled_v,
               sem):
    w = lax.axis_index("s") * N_CORES + lax.axis_index("c")
    base = w * IDX_PER_W

    pltpu.sync_copy(i2_hbm.at[pl.ds(base, IDX_PER_W)], idx_v)
    pltpu.sync_copy(cb_hbm.at[pl.ds(base, IDX_PER_W)],
                    cb_v.at[pl.ds(0, IDX_PER_W)])

    def sample_body(i, carry):
        off = L * i
        cp0 = pltpu.async_copy(
            t2_hbm.at[idx_v.at[pl.ds(off, G0)]],
            rows_v.at[pl.ds(0, G0)], sem)
        cp1 = pltpu.async_copy(
            t2_hbm.at[idx_v.at[pl.ds(off + G0, G1)]],
            rows_v.at[pl.ds(G0, G1)], sem)
        cp0.wait()
        cp1.wait()

        def accum_rows(r0, n, acc):
            cbv = cb_v[pl.ds(off + r0, 16)]
            accs = list(acc)
            for j in range(n):
                cb = pl.multiple_of(cbv[j], 64)
                for c in range(4):
                    accs[c] = accs[c] + rows_v[r0 + j, pl.ds(cb + 16 * c, 16)]
            return tuple(accs)

        def row_body(g, acc):
            return accum_rows(16 * g, 16, acc)

        acc = lax.fori_loop(
            0, L // 16, row_body,
            tuple(jnp.zeros((16,), jnp.float32) for _ in range(4)))
        acc = accum_rows(16 * (L // 16), L % 16, acc)
        inv = jnp.float32(1.0 / L)
        for c in range(4):
            pooled_v[pl.ds(D * i + 16 * c, 16)] = acc[c] * inv
        return carry

    lax.fori_loop(0, S_PER_W, sample_body, 0)
    pltpu.sync_copy(pooled_v, out_hbm.at[pl.ds(w * S_PER_W * D, S_PER_W * D)])


def _pool(i2, cb, t2):
    mesh = plsc.VectorSubcoreMesh(core_axis_name="c", subcore_axis_name="s")
    kern = functools.partial(
        pl.kernel,
        mesh=mesh,
        out_type=jax.ShapeDtypeStruct((B * D,), jnp.float32),
        scratch_types=[
            pltpu.VMEM((IDX_PER_W,), jnp.int32),
            pltpu.VMEM((IDX_PER_W + 16,), jnp.int32),
            pltpu.VMEM((L, 128), jnp.float32),
            pltpu.VMEM((S_PER_W * D,), jnp.float32),
            pltpu.SemaphoreType.DMA,
        ],
    )(_pool_body)
    return kern(i2, cb, t2)


def _mlp_body(h_ref, w1_ref, b1_ref, w2_ref, b2_ref, out_ref):
    h = h_ref[...]
    z = jnp.maximum(
        lax.dot(h, w1_ref[...], preferred_element_type=jnp.float32)
        + b1_ref[...], 0.0)
    out_ref[...] = (
        lax.dot(z, w2_ref[...], preferred_element_type=jnp.float32)
        + b2_ref[...])


def kernel(x, table, W1, b1, W2, b2):
    t2 = table.reshape(table.shape[0] // 2, 2 * D)
    xf = x.reshape(-1)
    i2 = xf >> 1
    cb = (xf & 1) << 6

    pooled = _pool(i2, cb, t2).reshape(B, D)

    w2p = jnp.pad(W2, ((0, 0), (0, 128 - W2.shape[1])))
    b2p = jnp.pad(b2, (0, 128 - b2.shape[0])).reshape(1, 128)
    outp = pl.pallas_call(
        _mlp_body,
        out_shape=jax.ShapeDtypeStruct((B, 128), jnp.float32),
    )(pooled, W1, b1.reshape(1, D), w2p, b2p)
    return outp[:, :W2.shape[1]]


# speedup vs baseline: 1.4044x; 1.4044x over previous
import functools

import jax
import jax.numpy as jnp
from jax import lax
from jax.experimental import pallas as pl
from jax.experimental.pallas import tpu as pltpu
from jax.experimental.pallas import tpu_sc as plsc

B = 4096
L = 200
D = 64
N_CORES = 2
N_SUBCORES = 16
N_WORKERS = N_CORES * N_SUBCORES
S_PER_W = B // N_WORKERS
IDX_PER_W = S_PER_W * L
G0 = 128
G1 = L - G0


def _pool_body(i2_hbm, cb_hbm, t2_hbm, out_hbm, idx_v, cb_v, rows_v, pooled_v,
               sem):
    w = lax.axis_index("s") * N_CORES + lax.axis_index("c")
    base = w * IDX_PER_W

    pltpu.sync_copy(i2_hbm.at[pl.ds(base, IDX_PER_W)], idx_v)
    pltpu.sync_copy(cb_hbm.at[pl.ds(base, IDX_PER_W)],
                    cb_v.at[pl.ds(0, IDX_PER_W)])

    def sample_body(i, carry):
        off = L * i
        cp0 = pltpu.async_copy(
            t2_hbm.at[idx_v.at[pl.ds(off, G0)]],
            rows_v.at[pl.ds(0, G0)], sem)
        cp1 = pltpu.async_copy(
            t2_hbm.at[idx_v.at[pl.ds(off + G0, G1)]],
            rows_v.at[pl.ds(G0, G1)], sem)
        cp0.wait()
        cp1.wait()

        def accum_rows(r0, n, acc):
            cbv = cb_v[pl.ds(off + r0, 16)]
            accs = list(acc)
            for j in range(n):
                cb = pl.multiple_of(cbv[j], 64)
                for c in range(4):
                    accs[c] = accs[c] + rows_v[r0 + j, pl.ds(cb + 16 * c, 16)]
            return tuple(accs)

        def row_body(g, acc):
            return accum_rows(16 * g, 16, acc)

        acc = lax.fori_loop(
            0, L // 16, row_body,
            tuple(jnp.zeros((16,), jnp.float32) for _ in range(4)))
        acc = accum_rows(16 * (L // 16), L % 16, acc)
        inv = jnp.float32(1.0 / L)
        for c in range(4):
            pooled_v[pl.ds(D * i + 16 * c, 16)] = acc[c] * inv
        return carry

    lax.fori_loop(0, S_PER_W, sample_body, 0)
    pltpu.sync_copy(pooled_v, out_hbm.at[pl.ds(w * S_PER_W * D, S_PER_W * D)])


def _pool(i2, cb, t2):
    mesh = plsc.VectorSubcoreMesh(core_axis_name="c", subcore_axis_name="s")
    kern = functools.partial(
        pl.kernel,
        mesh=mesh,
        out_type=jax.ShapeDtypeStruct((B * D,), jnp.float32),
        scratch_types=[
            pltpu.VMEM((IDX_PER_W,), jnp.int32),
            pltpu.VMEM((IDX_PER_W + 16,), jnp.int32),
            pltpu.VMEM((L, 128), jnp.float32),
            pltpu.VMEM((S_PER_W * D,), jnp.float32),
            pltpu.SemaphoreType.DMA,
        ],
    )(_pool_body)
    return kern(i2, cb, t2)


N_EMB_TOTAL = 1000000
REPACK_R = 2048
REPACK_GRID = 245
N_HALF = REPACK_R * REPACK_GRID


def _repack_body(a_ref, b_ref, out_ref):
    out_ref[:, 0:D] = a_ref[...].T
    out_ref[:, D:2 * D] = b_ref[...].T


def _repack(tt):
    return pl.pallas_call(
        _repack_body,
        grid=(REPACK_GRID,),
        in_specs=[
            pl.BlockSpec((D, REPACK_R), lambda k: (0, k)),
            pl.BlockSpec(
                (D, REPACK_R),
                lambda k: (0, jnp.minimum(k + REPACK_GRID,
                                          N_EMB_TOTAL // REPACK_R))),
        ],
        out_specs=pl.BlockSpec((REPACK_R, 2 * D), lambda k: (k, 0)),
        out_shape=jax.ShapeDtypeStruct((N_HALF, 2 * D), jnp.float32),
    )(tt, tt)


def _mlp_body(h_ref, w1_ref, b1_ref, w2_ref, b2_ref, out_ref):
    h = h_ref[...]
    z = jnp.maximum(
        lax.dot(h, w1_ref[...], preferred_element_type=jnp.float32)
        + b1_ref[...], 0.0)
    out_ref[...] = (
        lax.dot(z, w2_ref[...], preferred_element_type=jnp.float32)
        + b2_ref[...])


def kernel(x, table, W1, b1, W2, b2):
    t2 = _repack(table.T)
    xf = x.reshape(-1)
    half = xf >= N_HALF
    i2 = jnp.where(half, xf - N_HALF, xf)
    cb = jnp.where(half, D, 0)

    pooled = _pool(i2, cb, t2).reshape(B, D)

    w2p = jnp.pad(W2, ((0, 0), (0, 128 - W2.shape[1])))
    b2p = jnp.pad(b2, (0, 128 - b2.shape[0])).reshape(1, 128)
    outp = pl.pallas_call(
        _mlp_body,
        out_shape=jax.ShapeDtypeStruct((B, 128), jnp.float32),
    )(pooled, W1, b1.reshape(1, D), w2p, b2p)
    return outp[:, :W2.shape[1]]


# speedup vs baseline: 1.6412x; 1.1686x over previous
import functools

import jax
import jax.numpy as jnp
from jax import lax
from jax.experimental import pallas as pl
from jax.experimental.pallas import tpu as pltpu
from jax.experimental.pallas import tpu_sc as plsc

B = 4096
L = 200
D = 64
N_CORES = 2
N_SUBCORES = 16
N_WORKERS = N_CORES * N_SUBCORES
S_PER_W = B // N_WORKERS
IDX_PER_W = S_PER_W * L
G0 = 128
G1 = L - G0


def _pool_body(i2_hbm, cb_hbm, t2_hbm, out_hbm, idx_v, cb_v, rows_v0, rows_v1,
               pooled_v, sem0, sem1):
    w = lax.axis_index("s") * N_CORES + lax.axis_index("c")
    base = w * IDX_PER_W

    pltpu.sync_copy(i2_hbm.at[pl.ds(base, IDX_PER_W)], idx_v)
    pltpu.sync_copy(cb_hbm.at[pl.ds(base, IDX_PER_W)],
                    cb_v.at[pl.ds(0, IDX_PER_W)])

    def issue(i, rows_v, sem):
        off = L * i
        pltpu.async_copy(
            t2_hbm.at[idx_v.at[pl.ds(off, G0)]],
            rows_v.at[pl.ds(0, G0)], sem)
        pltpu.async_copy(
            t2_hbm.at[idx_v.at[pl.ds(off + G0, G1)]],
            rows_v.at[pl.ds(G0, G1)], sem)

    def drain(rows_v, sem):
        pltpu.make_async_copy(
            t2_hbm.at[pl.ds(0, G0)], rows_v.at[pl.ds(0, G0)], sem).wait()
        pltpu.make_async_copy(
            t2_hbm.at[pl.ds(0, G1)], rows_v.at[pl.ds(G0, G1)], sem).wait()

    def accum_sample(i, rows_v):
        off = L * i

        def accum_rows(r0, n, acc):
            cbv = cb_v[pl.ds(off + r0, 16)]
            accs = list(acc)
            for j in range(n):
                cb = pl.multiple_of(cbv[j], 64)
                for c in range(4):
                    accs[c] = accs[c] + rows_v[r0 + j, pl.ds(cb + 16 * c, 16)]
            return tuple(accs)

        acc = lax.fori_loop(
            0, L // 16, lambda g, a: accum_rows(16 * g, 16, a),
            tuple(jnp.zeros((16,), jnp.float32) for _ in range(4)))
        acc = accum_rows(16 * (L // 16), L % 16, acc)
        inv = jnp.float32(1.0 / L)
        for c in range(4):
            pooled_v[pl.ds(D * i + 16 * c, 16)] = acc[c] * inv

    issue(0, rows_v0, sem0)
    issue(1, rows_v1, sem1)

    def pair_body(k, carry):
        i = 2 * k
        drain(rows_v0, sem0)
        accum_sample(i, rows_v0)

        @pl.when(i + 2 < S_PER_W)
        def _():
            issue(i + 2, rows_v0, sem0)

        drain(rows_v1, sem1)
        accum_sample(i + 1, rows_v1)

        @pl.when(i + 3 < S_PER_W)
        def _():
            issue(i + 3, rows_v1, sem1)

        return carry

    lax.fori_loop(0, S_PER_W // 2, pair_body, 0)
    pltpu.sync_copy(pooled_v, out_hbm.at[pl.ds(w * S_PER_W * D, S_PER_W * D)])


def _pool(i2, cb, t2):
    mesh = plsc.VectorSubcoreMesh(core_axis_name="c", subcore_axis_name="s")
    kern = functools.partial(
        pl.kernel,
        mesh=mesh,
        out_type=jax.ShapeDtypeStruct((B * D,), jnp.float32),
        scratch_types=[
            pltpu.VMEM((IDX_PER_W,), jnp.int32),
            pltpu.VMEM((IDX_PER_W + 16,), jnp.int32),
            pltpu.VMEM((L, 128), jnp.float32),
            pltpu.VMEM((L, 128), jnp.float32),
            pltpu.VMEM((S_PER_W * D,), jnp.float32),
            pltpu.SemaphoreType.DMA,
            pltpu.SemaphoreType.DMA,
        ],
    )(_pool_body)
    return kern(i2, cb, t2)


N_EMB_TOTAL = 1000000
REPACK_R = 2048
REPACK_GRID = 245
N_HALF = REPACK_R * REPACK_GRID


def _repack_body(a_ref, b_ref, out_ref):
    out_ref[:, 0:D] = a_ref[...].T
    out_ref[:, D:2 * D] = b_ref[...].T


def _repack(tt):
    return pl.pallas_call(
        _repack_body,
        grid=(REPACK_GRID,),
        in_specs=[
            pl.BlockSpec((D, REPACK_R), lambda k: (0, k)),
            pl.BlockSpec(
                (D, REPACK_R),
                lambda k: (0, jnp.minimum(k + REPACK_GRID,
                                          N_EMB_TOTAL // REPACK_R))),
        ],
        out_specs=pl.BlockSpec((REPACK_R, 2 * D), lambda k: (k, 0)),
        out_shape=jax.ShapeDtypeStruct((N_HALF, 2 * D), jnp.float32),
    )(tt, tt)


def _mlp_body(h_ref, w1_ref, b1_ref, w2_ref, b2_ref, out_ref):
    h = h_ref[...]
    z = jnp.maximum(
        lax.dot(h, w1_ref[...], preferred_element_type=jnp.float32)
        + b1_ref[...], 0.0)
    out_ref[...] = (
        lax.dot(z, w2_ref[...], preferred_element_type=jnp.float32)
        + b2_ref[...])


def kernel(x, table, W1, b1, W2, b2):
    t2 = _repack(table.T)
    xf = x.reshape(-1)
    half = xf >= N_HALF
    i2 = jnp.where(half, xf - N_HALF, xf)
    cb = jnp.where(half, D, 0)

    pooled = _pool(i2, cb, t2).reshape(B, D)

    w2p = jnp.pad(W2, ((0, 0), (0, 128 - W2.shape[1])))
    b2p = jnp.pad(b2, (0, 128 - b2.shape[0])).reshape(1, 128)
    outp = pl.pallas_call(
        _mlp_body,
        out_shape=jax.ShapeDtypeStruct((B, 128), jnp.float32),
    )(pooled, W1, b1.reshape(1, D), w2p, b2p)
    return outp[:, :W2.shape[1]]


# speedup vs baseline: 1.8558x; 1.1307x over previous
import functools

import jax
import jax.numpy as jnp
from jax import lax
from jax.experimental import pallas as pl
from jax.experimental.pallas import tpu as pltpu
from jax.experimental.pallas import tpu_sc as plsc

B = 4096
L = 200
D = 64
N_CORES = 2
N_SUBCORES = 16
N_WORKERS = N_CORES * N_SUBCORES
S_PER_W = B // N_WORKERS
IDX_PER_W = S_PER_W * L
G0 = 128
G1 = L - G0


def _pool_body(i2_hbm, cb_hbm, t2_hbm, out_hbm, idx_v, cb_v, rows_v0, rows_v1,
               pooled_v, sem0, sem1):
    w = lax.axis_index("s") * N_CORES + lax.axis_index("c")
    base = w * IDX_PER_W

    pltpu.sync_copy(i2_hbm.at[pl.ds(base, IDX_PER_W)], idx_v)
    pltpu.sync_copy(cb_hbm.at[pl.ds(base, IDX_PER_W)],
                    cb_v.at[pl.ds(0, IDX_PER_W)])

    def issue(i, rows_v, sem):
        off = L * i
        pltpu.async_copy(
            t2_hbm.at[idx_v.at[pl.ds(off, G0)]],
            rows_v.at[pl.ds(0, G0)], sem)
        pltpu.async_copy(
            t2_hbm.at[idx_v.at[pl.ds(off + G0, G1)]],
            rows_v.at[pl.ds(G0, G1)], sem)

    def drain(rows_v, sem):
        pltpu.make_async_copy(
            t2_hbm.at[pl.ds(0, G0)], rows_v.at[pl.ds(0, G0)], sem).wait()
        pltpu.make_async_copy(
            t2_hbm.at[pl.ds(0, G1)], rows_v.at[pl.ds(G0, G1)], sem).wait()

    def accum_sample(i, rows_v):
        off = L * i

        def accum_rows(r0, n, acc):
            cbv = cb_v[pl.ds(off + r0, 16)]
            accs = list(acc)
            for j in range(n):
                cb = pl.multiple_of(cbv[j], 64)
                for c in range(4):
                    accs[c] = accs[c] + rows_v[r0 + j, pl.ds(cb + 16 * c, 16)]
            return tuple(accs)

        acc = lax.fori_loop(
            0, L // 16, lambda g, a: accum_rows(16 * g, 16, a),
            tuple(jnp.zeros((16,), jnp.float32) for _ in range(4)))
        acc = accum_rows(16 * (L // 16), L % 16, acc)
        inv = jnp.float32(1.0 / L)
        for c in range(4):
            pooled_v[pl.ds(D * i + 16 * c, 16)] = acc[c] * inv

    issue(0, rows_v0, sem0)
    issue(1, rows_v1, sem1)

    def pair_body(k, carry):
        i = 2 * k
        drain(rows_v0, sem0)
        accum_sample(i, rows_v0)

        @pl.when(i + 2 < S_PER_W)
        def _():
            issue(i + 2, rows_v0, sem0)

        drain(rows_v1, sem1)
        accum_sample(i + 1, rows_v1)

        @pl.when(i + 3 < S_PER_W)
        def _():
            issue(i + 3, rows_v1, sem1)

        return carry

    lax.fori_loop(0, S_PER_W // 2, pair_body, 0)
    pltpu.sync_copy(pooled_v, out_hbm.at[pl.ds(w * S_PER_W * D, S_PER_W * D)])


def _pool(i2, cb, t2):
    mesh = plsc.VectorSubcoreMesh(core_axis_name="c", subcore_axis_name="s")
    kern = functools.partial(
        pl.kernel,
        mesh=mesh,
        out_type=jax.ShapeDtypeStruct((B * D,), jnp.float32),
        scratch_types=[
            pltpu.VMEM((IDX_PER_W,), jnp.int32),
            pltpu.VMEM((IDX_PER_W + 16,), jnp.int32),
            pltpu.VMEM((L, 128), jnp.float32),
            pltpu.VMEM((L, 128), jnp.float32),
            pltpu.VMEM((S_PER_W * D,), jnp.float32),
            pltpu.SemaphoreType.DMA,
            pltpu.SemaphoreType.DMA,
        ],
    )(_pool_body)
    return kern(i2, cb, t2)


N_EMB_TOTAL = 1000000
REPACK_R = 4096
REPACK_GRID = 123
N_HALF = REPACK_R * REPACK_GRID


def _repack_body(a_ref, b_ref, eye_ref, out_ref):
    eye = eye_ref[...]
    at = lax.dot_general(a_ref[...], eye, (((0,), (0,)), ((), ())),
                         preferred_element_type=jnp.float32)
    bt = lax.dot_general(b_ref[...], eye, (((0,), (0,)), ((), ())),
                         preferred_element_type=jnp.float32)
    out_ref[...] = jnp.concatenate([at, bt], axis=1)


def _repack(tt):
    return pl.pallas_call(
        _repack_body,
        grid=(REPACK_GRID,),
        in_specs=[
            pl.BlockSpec((D, REPACK_R), lambda k: (0, k)),
            pl.BlockSpec(
                (D, REPACK_R),
                lambda k: (0, jnp.minimum(k + REPACK_GRID,
                                          N_EMB_TOTAL // REPACK_R))),
            pl.BlockSpec((D, D), lambda k: (0, 0)),
        ],
        out_specs=pl.BlockSpec((REPACK_R, 2 * D), lambda k: (k, 0)),
        out_shape=jax.ShapeDtypeStruct((N_HALF, 2 * D), jnp.float32),
    )(tt, tt, jnp.eye(D, dtype=jnp.float32))


def _mlp_body(h_ref, w1_ref, b1_ref, w2_ref, b2_ref, out_ref):
    h = h_ref[...]
    z = jnp.maximum(
        lax.dot(h, w1_ref[...], preferred_element_type=jnp.float32)
        + b1_ref[...], 0.0)
    out_ref[...] = (
        lax.dot(z, w2_ref[...], preferred_element_type=jnp.float32)
        + b2_ref[...])


def kernel(x, table, W1, b1, W2, b2):
    t2 = _repack(table.T)
    xf = x.reshape(-1)
    half = xf >= N_HALF
    i2 = jnp.where(half, xf - N_HALF, xf)
    cb = jnp.where(half, D, 0)

    pooled = _pool(i2, cb, t2).reshape(B, D)

    w2p = jnp.pad(W2, ((0, 0), (0, 128 - W2.shape[1])))
    b2p = jnp.pad(b2, (0, 128 - b2.shape[0])).reshape(1, 128)
    outp = pl.pallas_call(
        _mlp_body,
        out_shape=jax.ShapeDtypeStruct((B, 128), jnp.float32),
    )(pooled, W1, b1.reshape(1, D), w2p, b2p)
    return outp[:, :W2.shape[1]]
